# baseline (speedup 1.0000x reference)
import functools

import jax
import jax.numpy as jnp
from jax import lax
from jax.experimental import pallas as pl
from jax.experimental.pallas import tpu as pltpu
from jax.experimental.pallas import tpu_sc as plsc

NC = 2
NS = 16
CHUNK = 1024
KSUB = CHUNK // 128
GPC = 8


def _sc_edge_accumulate(o_t, srcr, dstr, wr, zrows, *, n_pad, e_pad):
    cpt = e_pad // (NC * NS) // CHUNK
    rpt = n_pad // NS
    gpt = rpt // 128
    nq = n_pad // 128
    mesh = plsc.VectorSubcoreMesh(core_axis_name="c", subcore_axis_name="s")

    @functools.partial(
        pl.kernel,
        out_type=[jax.ShapeDtypeStruct((nq, 8, 128), jnp.float32),
                  jax.ShapeDtypeStruct((nq, 8, 128), jnp.float32)],
        mesh=mesh,
        compiler_params=pltpu.CompilerParams(use_tc_tiling_on_sc=False,
                                             needs_layout_passes=False),
        scratch_types=[
            pltpu.VMEM((KSUB, 128), jnp.int32),
            pltpu.VMEM((KSUB, 128), jnp.int32),
            pltpu.VMEM((KSUB, 128), jnp.int32),
            pltpu.VMEM((KSUB, 128), jnp.int32),
            pltpu.VMEM((CHUNK,), jnp.float32),
            pltpu.VMEM((CHUNK,), jnp.float32),
            pltpu.VMEM((CHUNK, 8), jnp.float32),
            pltpu.VMEM((CHUNK, 8), jnp.float32),
            pltpu.VMEM((CHUNK, 8), jnp.float32),
            pltpu.VMEM((CHUNK, 8), jnp.float32),
            pltpu.VMEM((KSUB, 128), jnp.int32),
            pltpu.VMEM((KSUB, 128), jnp.int32),
            pltpu.VMEM((GPC, 8, 128), jnp.float32),
            pltpu.VMEM_SHARED((n_pad, 8), jnp.float32),
            pltpu.SemaphoreType.DMA,
            pltpu.SemaphoreType.DMA,
            pltpu.SemaphoreType.DMA,
            pltpu.SemaphoreType.DMA,
            pltpu.SemaphoreType.DMA,
            pltpu.SemaphoreType.DMA,
        ],
    )
    def sc_kernel(o_t_hbm, src_hbm, dst_hbm, w_hbm, z_hbm, out0_hbm, out1_hbm,
                  src_v0, src_v1, dst_v0, dst_v1, w_v0, w_v1,
                  rows_v0, rows_v1, contrib_v0, contrib_v1, dsts_v0, dsts_v1,
                  stage_v, acc_sh, lsem0, lsem1, gsem0, gsem1, ssem0, ssem1):
        src_b = (src_v0, src_v1)
        dst_b = (dst_v0, dst_v1)
        w_b = (w_v0, w_v1)
        rows_b = (rows_v0, rows_v1)
        contrib_b = (contrib_v0, contrib_v1)
        dsts_b = (dsts_v0, dsts_v1)
        lsem = (lsem0, lsem1)
        gsem = (gsem0, gsem1)
        ssem = (ssem0, ssem1)
        rows_v = rows_v0

        cid = lax.axis_index("c")
        sid = lax.axis_index("s")
        tile = cid * NS + sid
        r0 = sid * rpt
        base_row = tile * (cpt * KSUB)
        iota16 = lax.iota(jnp.int32, 16)
        cols = [jnp.full((16,), b, jnp.int32) for b in range(8)]

        def issue_l(ci, p):
            row0 = base_row + ci * KSUB
            e0 = (base_row + ci * KSUB) * 128
            pltpu.async_copy(src_hbm.at[pl.ds(row0, KSUB)], src_b[p], lsem[p])
            pltpu.async_copy(dst_hbm.at[pl.ds(row0, KSUB)], dst_b[p], lsem[p])
            pltpu.async_copy(w_hbm.at[pl.ds(e0, CHUNK)], w_b[p], lsem[p])

        def wait_l(p):
            pltpu.make_async_copy(src_hbm.at[pl.ds(0, KSUB)], src_b[p], lsem[p]).wait()
            pltpu.make_async_copy(dst_hbm.at[pl.ds(0, KSUB)], dst_b[p], lsem[p]).wait()
            pltpu.make_async_copy(w_hbm.at[pl.ds(0, CHUNK)], w_b[p], lsem[p]).wait()

        def issue_g(p):
            for j in range(KSUB):
                pltpu.async_copy(o_t_hbm.at[src_b[p].at[j]],
                                 rows_b[p].at[pl.ds(j * 128, 128)], gsem[p])

        def wait_g(p):
            for j in range(KSUB):
                pltpu.make_async_copy(o_t_hbm.at[src_b[p].at[j]],
                                      rows_b[p].at[pl.ds(j * 128, 128)],
                                      gsem[p]).wait()

        def issue_s(p):
            for j in range(KSUB):
                pltpu.async_copy(contrib_b[p].at[pl.ds(j * 128, 128)],
                                 acc_sh.at[dsts_b[p].at[j]], ssem[p], add=True)

        def wait_s(p):
            for j in range(KSUB):
                pltpu.make_async_copy(contrib_b[p].at[pl.ds(j * 128, 128)],
                                      acc_sh.at[dsts_b[p].at[j]],
                                      ssem[p]).wait()

        def compute(p):
            for j in range(KSUB):
                for q in range(8):
                    sl = pl.ds(q * 16, 16)
                    dsts_b[p][j, sl] = dst_b[p][j, sl]

            @plsc.parallel_loop(0, CHUNK // 16, unroll=8)
            def _(g):
                base = g * 16
                wvec = w_b[p][pl.ds(base, 16)]
                ridx = iota16 + base
                for b in range(8):
                    vals = plsc.load_gather(rows_b[p], [ridx, cols[b]])
                    plsc.store_scatter(contrib_b[p], [ridx, cols[b]],
                                       vals * wvec)

        def step(ci, p):
            q = 1 - p

            @pl.when(ci + 1 < cpt)
            def _():
                wait_l(q)
                issue_g(q)

            wait_g(p)

            @pl.when(ci >= 1)
            def _():
                wait_s(q)

            compute(p)
            issue_s(p)

            @pl.when(ci + 2 < cpt)
            def _():
                issue_l(ci + 2, p)

        issue_l(0, 0)
        pltpu.sync_copy(z_hbm.at[pl.ds(r0, rpt)], acc_sh.at[pl.ds(r0, rpt)])
        wait_l(0)
        issue_g(0)
        if cpt > 1:
            issue_l(1, 1)
        plsc.subcore_barrier()

        def pair_body(k, carry):
            step(2 * k, 0)
            step(2 * k + 1, 1)
            return carry

        lax.fori_loop(0, cpt // 2, pair_body, 0)
        if cpt % 2:
            step(jnp.int32(cpt - 1), (cpt - 1) % 2)
        wait_s((cpt - 1) % 2)
        plsc.subcore_barrier()

        q0 = sid * gpt

        def emit(rbody_ci, ngrp):
            pltpu.sync_copy(
                acc_sh.at[pl.ds(r0 + rbody_ci * (GPC * 128), ngrp * 128)],
                rows_v.at[pl.ds(0, ngrp * 128)])
            for g in range(ngrp):
                for b in range(8):
                    for q in range(8):
                        ridx = iota16 + (g * 128 + q * 16)
                        vals = plsc.load_gather(rows_v, [ridx, cols[b]])
                        stage_v[g, b, pl.ds(q * 16, 16)] = vals

        def rbody(ci, carry):
            emit(ci, GPC)

            @pl.when(cid == 0)
            def _():
                pltpu.sync_copy(stage_v,
                                out0_hbm.at[pl.ds(q0 + ci * GPC, GPC)])

            @pl.when(cid == 1)
            def _():
                pltpu.sync_copy(stage_v,
                                out1_hbm.at[pl.ds(q0 + ci * GPC, GPC)])
            return carry

        lax.fori_loop(0, gpt // GPC, rbody, 0)
        rem = gpt % GPC
        if rem:
            emit(gpt // GPC, rem)

            @pl.when(cid == 0)
            def _():
                pltpu.sync_copy(stage_v.at[pl.ds(0, rem)],
                                out0_hbm.at[pl.ds(q0 + gpt - rem, rem)])

            @pl.when(cid == 1)
            def _():
                pltpu.sync_copy(stage_v.at[pl.ds(0, rem)],
                                out1_hbm.at[pl.ds(q0 + gpt - rem, rem)])

    return sc_kernel(o_t, srcr, dstr, wr, zrows)


BLK = 4096
BQ = BLK // 128


def _epilogue_body(q0_ref, q1_ref, chem_ref, thr_ref, dec_ref,
                   new_o_ref, new_e_ref):
    for t in range(BQ):
        g = q0_ref[t] + q1_ref[t]
        sl = pl.ds(t * 128, 128)
        s = jnp.clip(chem_ref[:, sl] + g, -10.0, 10.0)
        th = thr_ref[:, sl]
        new_o = jnp.maximum(s - th, 0.0)
        gt = s > th
        eq = s == 0.0
        new_o_ref[:, sl] = new_o
        new_e_ref[:, sl] = jnp.where(gt, new_o,
                                     jnp.where(eq, -dec_ref[:, sl], s))


def _epilogue(q0, q1, chem, thr2, dec2):
    b, n = chem.shape
    grid = (-(-n // BLK),)
    row_spec = pl.BlockSpec((b, BLK), lambda i: (0, i))
    one_spec = pl.BlockSpec((1, BLK), lambda i: (0, i))
    q_spec = pl.BlockSpec((BQ, 8, 128), lambda i: (i, 0, 0))
    return pl.pallas_call(
        _epilogue_body,
        grid=grid,
        in_specs=[q_spec, q_spec, row_spec, one_spec, one_spec],
        out_specs=[row_spec, row_spec],
        out_shape=[
            jax.ShapeDtypeStruct((b, n), jnp.float32),
            jax.ShapeDtypeStruct((b, n), jnp.float32),
        ],
    )(q0, q1, chem, thr2, dec2)


def kernel(chem_influence, o_pre, w, threshold, decay, src, dst):
    b, n = chem_influence.shape
    e = src.shape[0]
    per = NC * NS * CHUNK
    cpt = -(-e // per)
    e_pad = per * cpt
    pad = e_pad - e
    if pad:
        src = jnp.concatenate([src, jnp.zeros((pad,), jnp.int32)])
        dst = jnp.concatenate([dst, jnp.zeros((pad,), jnp.int32)])
        w = jnp.concatenate([w, jnp.zeros((pad,), jnp.float32)])
    srcr = src.reshape(e_pad // 128, 128)
    dstr = dst.reshape(e_pad // 128, 128)
    wr = w
    n_pad = -(-n // (NS * 128)) * (NS * 128)
    o_t = jnp.abs(o_pre).T
    zrows = jnp.zeros((n_pad, 8), jnp.float32)

    p0, p1 = _sc_edge_accumulate(o_t, srcr, dstr, wr, zrows,
                                 n_pad=n_pad, e_pad=e_pad)
    thr2 = threshold.reshape(1, n)
    dec2 = decay.reshape(1, n)
    new_o, new_e = _epilogue(p0, p1, chem_influence, thr2, dec2)
    return (new_o, (new_e, new_o))


# speedup vs baseline: 1.0261x; 1.0261x over previous
import functools

import jax
import jax.numpy as jnp
from jax import lax
from jax.experimental import pallas as pl
from jax.experimental.pallas import tpu as pltpu
from jax.experimental.pallas import tpu_sc as plsc

NC = 2
NS = 16
CHUNK = 1024
KSUB = CHUNK // 128
GPC = 8


def _sc_edge_accumulate(o_t, srcr, dstr, wr, zrows, *, n_pad, e_pad):
    cpt = e_pad // (NC * NS) // CHUNK
    rpt = n_pad // NS
    gpt = rpt // 128
    nq = n_pad // 128
    mesh = plsc.VectorSubcoreMesh(core_axis_name="c", subcore_axis_name="s")

    @functools.partial(
        pl.kernel,
        out_type=[jax.ShapeDtypeStruct((nq, 8, 128), jnp.float32),
                  jax.ShapeDtypeStruct((nq, 8, 128), jnp.float32)],
        mesh=mesh,
        compiler_params=pltpu.CompilerParams(use_tc_tiling_on_sc=False,
                                             needs_layout_passes=False),
        scratch_types=[
            pltpu.VMEM((KSUB, 128), jnp.int32),
            pltpu.VMEM((KSUB, 128), jnp.int32),
            pltpu.VMEM((KSUB, 128), jnp.int32),
            pltpu.VMEM((KSUB, 128), jnp.int32),
            pltpu.VMEM((CHUNK,), jnp.float32),
            pltpu.VMEM((CHUNK,), jnp.float32),
            pltpu.VMEM((CHUNK, 8), jnp.float32),
            pltpu.VMEM((CHUNK, 8), jnp.float32),
            pltpu.VMEM((CHUNK, 8), jnp.float32),
            pltpu.VMEM((CHUNK, 8), jnp.float32),
            pltpu.VMEM((KSUB, 128), jnp.int32),
            pltpu.VMEM((KSUB, 128), jnp.int32),
            pltpu.VMEM((GPC, 8, 128), jnp.float32),
            pltpu.VMEM_SHARED((n_pad, 8), jnp.float32),
            pltpu.SemaphoreType.DMA,
            pltpu.SemaphoreType.DMA,
            pltpu.SemaphoreType.DMA,
            pltpu.SemaphoreType.DMA,
            pltpu.SemaphoreType.DMA,
            pltpu.SemaphoreType.DMA,
        ],
    )
    def sc_kernel(o_t_hbm, src_hbm, dst_hbm, w_hbm, z_hbm, out0_hbm, out1_hbm,
                  src_v0, src_v1, dst_v0, dst_v1, w_v0, w_v1,
                  rows_v0, rows_v1, contrib_v0, contrib_v1, dsts_v0, dsts_v1,
                  stage_v, acc_sh, lsem0, lsem1, gsem0, gsem1, ssem0, ssem1):
        src_b = (src_v0, src_v1)
        dst_b = (dst_v0, dst_v1)
        w_b = (w_v0, w_v1)
        rows_b = (rows_v0, rows_v1)
        contrib_b = (contrib_v0, contrib_v1)
        dsts_b = (dsts_v0, dsts_v1)
        lsem = (lsem0, lsem1)
        gsem = (gsem0, gsem1)
        ssem = (ssem0, ssem1)
        rows_v = rows_v0

        cid = lax.axis_index("c")
        sid = lax.axis_index("s")
        tile = cid * NS + sid
        r0 = sid * rpt
        base_row = tile * (cpt * KSUB)
        iota16 = lax.iota(jnp.int32, 16)
        cols = [jnp.full((16,), b, jnp.int32) for b in range(8)]

        def issue_l(ci, p):
            row0 = base_row + ci * KSUB
            e0 = (base_row + ci * KSUB) * 128
            pltpu.async_copy(src_hbm.at[pl.ds(row0, KSUB)], src_b[p], lsem[p])
            pltpu.async_copy(dst_hbm.at[pl.ds(row0, KSUB)], dst_b[p], lsem[p])
            pltpu.async_copy(w_hbm.at[pl.ds(e0, CHUNK)], w_b[p], lsem[p])

        def wait_l(p):
            pltpu.make_async_copy(src_hbm.at[pl.ds(0, KSUB)], src_b[p], lsem[p]).wait()
            pltpu.make_async_copy(dst_hbm.at[pl.ds(0, KSUB)], dst_b[p], lsem[p]).wait()
            pltpu.make_async_copy(w_hbm.at[pl.ds(0, CHUNK)], w_b[p], lsem[p]).wait()

        def issue_g(p):
            for j in range(KSUB):
                pltpu.async_copy(o_t_hbm.at[src_b[p].at[j]],
                                 rows_b[p].at[pl.ds(j * 128, 128)], gsem[p])

        def wait_g(p):
            for j in range(KSUB):
                pltpu.make_async_copy(o_t_hbm.at[src_b[p].at[j]],
                                      rows_b[p].at[pl.ds(j * 128, 128)],
                                      gsem[p]).wait()

        def issue_s(p):
            for j in range(KSUB):
                pltpu.async_copy(contrib_b[p].at[pl.ds(j * 128, 128)],
                                 acc_sh.at[dsts_b[p].at[j]], ssem[p], add=True)

        def wait_s(p):
            for j in range(KSUB):
                pltpu.make_async_copy(contrib_b[p].at[pl.ds(j * 128, 128)],
                                      acc_sh.at[dsts_b[p].at[j]],
                                      ssem[p]).wait()

        def compute(p):
            for j in range(KSUB):
                for q in range(8):
                    sl = pl.ds(q * 16, 16)
                    dsts_b[p][j, sl] = dst_b[p][j, sl]

            @plsc.parallel_loop(0, CHUNK // 16, unroll=4)
            def _(g):
                base = g * 16
                wvec = w_b[p][pl.ds(base, 16)]
                ridx = iota16 + base
                for b in range(8):
                    vals = plsc.load_gather(rows_b[p], [ridx, cols[b]])
                    plsc.store_scatter(contrib_b[p], [ridx, cols[b]],
                                       vals * wvec)

        def step(ci, p):
            q = 1 - p

            @pl.when(ci + 1 < cpt)
            def _():
                wait_l(q)
                issue_g(q)

            wait_g(p)

            @pl.when(ci >= 1)
            def _():
                wait_s(q)

            compute(p)
            issue_s(p)

            @pl.when(ci + 2 < cpt)
            def _():
                issue_l(ci + 2, p)

        issue_l(0, 0)
        pltpu.sync_copy(z_hbm.at[pl.ds(r0, rpt)], acc_sh.at[pl.ds(r0, rpt)])
        wait_l(0)
        issue_g(0)
        if cpt > 1:
            issue_l(1, 1)
        plsc.subcore_barrier()

        def pair_body(k, carry):
            step(2 * k, 0)
            step(2 * k + 1, 1)
            return carry

        lax.fori_loop(0, cpt // 2, pair_body, 0)
        if cpt % 2:
            step(jnp.int32(cpt - 1), (cpt - 1) % 2)
        wait_s((cpt - 1) % 2)
        plsc.subcore_barrier()

        q0 = sid * gpt

        def emit(rbody_ci, ngrp):
            pltpu.sync_copy(
                acc_sh.at[pl.ds(r0 + rbody_ci * (GPC * 128), ngrp * 128)],
                rows_v.at[pl.ds(0, ngrp * 128)])
            for g in range(ngrp):
                for b in range(8):
                    for q in range(8):
                        ridx = iota16 + (g * 128 + q * 16)
                        vals = plsc.load_gather(rows_v, [ridx, cols[b]])
                        stage_v[g, b, pl.ds(q * 16, 16)] = vals

        def rbody(ci, carry):
            emit(ci, GPC)

            @pl.when(cid == 0)
            def _():
                pltpu.sync_copy(stage_v,
                                out0_hbm.at[pl.ds(q0 + ci * GPC, GPC)])

            @pl.when(cid == 1)
            def _():
                pltpu.sync_copy(stage_v,
                                out1_hbm.at[pl.ds(q0 + ci * GPC, GPC)])
            return carry

        lax.fori_loop(0, gpt // GPC, rbody, 0)
        rem = gpt % GPC
        if rem:
            emit(gpt // GPC, rem)

            @pl.when(cid == 0)
            def _():
                pltpu.sync_copy(stage_v.at[pl.ds(0, rem)],
                                out0_hbm.at[pl.ds(q0 + gpt - rem, rem)])

            @pl.when(cid == 1)
            def _():
                pltpu.sync_copy(stage_v.at[pl.ds(0, rem)],
                                out1_hbm.at[pl.ds(q0 + gpt - rem, rem)])

    return sc_kernel(o_t, srcr, dstr, wr, zrows)


BLK = 4096
BQ = BLK // 128


def _epilogue_body(q0_ref, q1_ref, chem_ref, thr_ref, dec_ref,
                   new_o_ref, new_e_ref):
    for t in range(BQ):
        g = q0_ref[t] + q1_ref[t]
        sl = pl.ds(t * 128, 128)
        s = jnp.clip(chem_ref[:, sl] + g, -10.0, 10.0)
        th = thr_ref[:, sl]
        new_o = jnp.maximum(s - th, 0.0)
        gt = s > th
        eq = s == 0.0
        new_o_ref[:, sl] = new_o
        new_e_ref[:, sl] = jnp.where(gt, new_o,
                                     jnp.where(eq, -dec_ref[:, sl], s))


def _epilogue(q0, q1, chem, thr2, dec2):
    b, n = chem.shape
    grid = (-(-n // BLK),)
    row_spec = pl.BlockSpec((b, BLK), lambda i: (0, i))
    one_spec = pl.BlockSpec((1, BLK), lambda i: (0, i))
    q_spec = pl.BlockSpec((BQ, 8, 128), lambda i: (i, 0, 0))
    return pl.pallas_call(
        _epilogue_body,
        grid=grid,
        in_specs=[q_spec, q_spec, row_spec, one_spec, one_spec],
        out_specs=[row_spec, row_spec],
        out_shape=[
            jax.ShapeDtypeStruct((b, n), jnp.float32),
            jax.ShapeDtypeStruct((b, n), jnp.float32),
        ],
    )(q0, q1, chem, thr2, dec2)


def kernel(chem_influence, o_pre, w, threshold, decay, src, dst):
    b, n = chem_influence.shape
    e = src.shape[0]
    per = NC * NS * CHUNK
    cpt = -(-e // per)
    e_pad = per * cpt
    pad = e_pad - e
    if pad:
        src = jnp.concatenate([src, jnp.zeros((pad,), jnp.int32)])
        dst = jnp.concatenate([dst, jnp.zeros((pad,), jnp.int32)])
        w = jnp.concatenate([w, jnp.zeros((pad,), jnp.float32)])
    srcr = src.reshape(e_pad // 128, 128)
    dstr = dst.reshape(e_pad // 128, 128)
    wr = w
    n_pad = -(-n // (NS * 128)) * (NS * 128)
    o_t = jnp.abs(o_pre).T
    zrows = jnp.zeros((n_pad, 8), jnp.float32)

    p0, p1 = _sc_edge_accumulate(o_t, srcr, dstr, wr, zrows,
                                 n_pad=n_pad, e_pad=e_pad)
    thr2 = threshold.reshape(1, n)
    dec2 = decay.reshape(1, n)
    new_o, new_e = _epilogue(p0, p1, chem_influence, thr2, dec2)
    return (new_o, (new_e, new_o))
